# baseline (speedup 1.0000x reference)
import functools

import jax
import jax.numpy as jnp
from jax import lax
from jax.experimental import pallas as pl
from jax.experimental.pallas import tpu as pltpu
from jax.experimental.pallas import tpu_sc as plsc

_N = 4096
_D = 64
_NC = 2
_NS = 16
_NW = _NC * _NS
_RPW = _N // _NW
_L = 16


def _sc_rowdot(at_hbm, bt_hbm, out_hbm, a_v, b_v, o_v):
    wid = lax.axis_index("s") * _NC + lax.axis_index("c")
    base = wid * _RPW
    pltpu.sync_copy(at_hbm.at[:, pl.ds(base, _RPW)], a_v)
    pltpu.sync_copy(bt_hbm.at[:, pl.ds(base, _RPW)], b_v)
    for g in range(_RPW // _L):
        sl = pl.ds(g * _L, _L)
        acc = a_v[0, sl] * b_v[0, sl]
        for k in range(1, _D):
            acc = acc + a_v[k, sl] * b_v[k, sl]
        o_v[sl] = acc
    pltpu.sync_copy(o_v, out_hbm.at[pl.ds(base, _RPW)])


@jax.jit
def _sc_call(at, bt):
    return pl.kernel(
        _sc_rowdot,
        out_type=jax.ShapeDtypeStruct((_N,), jnp.float32),
        mesh=plsc.VectorSubcoreMesh(core_axis_name="c", subcore_axis_name="s"),
        scratch_types=[
            pltpu.VMEM((_D, _RPW), jnp.float32),
            pltpu.VMEM((_D, _RPW), jnp.float32),
            pltpu.VMEM((_RPW,), jnp.float32),
        ],
    )(at, bt)


def kernel(gum, gim):
    xui = _sc_call(gum.T, gim.T)
    return (xui, gum, gim)


# speedup vs baseline: 5.5507x; 5.5507x over previous
import jax
import jax.numpy as jnp
from jax.experimental import pallas as pl
from jax.experimental.pallas import tpu as pltpu

_N = 4096
_D = 64
_CH = 1024
_NCH = _N // _CH


def _pipe_kernel(at, bt, x, ao, bo, a_v, b_v, x_v, in_sem, out_sem, x_sem):
    def in_cps(i, buf):
        sl = pl.ds(i * _CH, _CH)
        return (
            pltpu.make_async_copy(at.at[:, sl], a_v.at[buf], in_sem.at[buf, 0]),
            pltpu.make_async_copy(bt.at[:, sl], b_v.at[buf], in_sem.at[buf, 1]),
        )

    def out_cps(i, buf):
        sl = pl.ds(i * _CH, _CH)
        return (
            pltpu.make_async_copy(a_v.at[buf], ao.at[:, sl], out_sem.at[buf, 0]),
            pltpu.make_async_copy(b_v.at[buf], bo.at[:, sl], out_sem.at[buf, 1]),
        )

    for c in in_cps(0, 0):
        c.start()
    for i in range(_NCH):
        buf = i % 2
        if i + 1 < _NCH:
            if i >= 1:
                for c in out_cps(i - 1, 1 - buf):
                    c.wait()
            for c in in_cps(i + 1, 1 - buf):
                c.start()
        for c in in_cps(i, buf):
            c.wait()
        a = a_v[buf]
        b = b_v[buf]
        x_v[pl.ds(i * _CH, _CH)] = jnp.sum(a * b, axis=0)
        for c in out_cps(i, buf):
            c.start()
    xc = pltpu.make_async_copy(x_v, x, x_sem)
    xc.start()
    for c in out_cps(_NCH - 2, _NCH % 2):
        c.wait()
    for c in out_cps(_NCH - 1, 1 - _NCH % 2):
        c.wait()
    xc.wait()


def kernel(gum, gim):
    n, d = gum.shape
    at = gum.T
    bt = gim.T
    x, aot, bot = pl.pallas_call(
        _pipe_kernel,
        in_specs=[
            pl.BlockSpec(memory_space=pl.ANY),
            pl.BlockSpec(memory_space=pl.ANY),
        ],
        out_specs=(
            pl.BlockSpec(memory_space=pl.ANY),
            pl.BlockSpec(memory_space=pl.ANY),
            pl.BlockSpec(memory_space=pl.ANY),
        ),
        out_shape=(
            jax.ShapeDtypeStruct((n,), jnp.float32),
            jax.ShapeDtypeStruct((d, n), jnp.float32),
            jax.ShapeDtypeStruct((d, n), jnp.float32),
        ),
        scratch_shapes=[
            pltpu.VMEM((2, _D, _CH), jnp.float32),
            pltpu.VMEM((2, _D, _CH), jnp.float32),
            pltpu.VMEM((_N,), jnp.float32),
            pltpu.SemaphoreType.DMA((2, 2)),
            pltpu.SemaphoreType.DMA((2, 2)),
            pltpu.SemaphoreType.DMA,
        ],
    )(at, bt)
    return (x, aot.T, bot.T)


# speedup vs baseline: 9.0557x; 1.6315x over previous
import jax
import jax.numpy as jnp
from jax.experimental import pallas as pl
from jax.experimental.pallas import tpu as pltpu

_N = 4096
_D = 64
_CH = 1024
_NCH = _N // _CH


def _pipe_kernel(at, bt, x, ao, bo, a_v, b_v, x_v, in_sem, out_sem, x_sem):
    def in_cps(i):
        sl = pl.ds(i * _CH, _CH)
        return (
            pltpu.make_async_copy(at.at[:, sl], a_v.at[:, sl], in_sem.at[i, 0]),
            pltpu.make_async_copy(bt.at[:, sl], b_v.at[:, sl], in_sem.at[i, 1]),
        )

    def out_cps(i):
        sl = pl.ds(i * _CH, _CH)
        return (
            pltpu.make_async_copy(a_v.at[:, sl], ao.at[:, sl], out_sem.at[i, 0]),
            pltpu.make_async_copy(b_v.at[:, sl], bo.at[:, sl], out_sem.at[i, 1]),
        )

    for i in range(_NCH):
        for c in in_cps(i):
            c.start()
    for i in range(_NCH):
        sl = pl.ds(i * _CH, _CH)
        for c in in_cps(i):
            c.wait()
        for c in out_cps(i):
            c.start()
        x_v[sl] = jnp.sum(a_v[:, sl] * b_v[:, sl], axis=0)
    xc = pltpu.make_async_copy(x_v, x, x_sem)
    xc.start()
    for i in range(_NCH):
        for c in out_cps(i):
            c.wait()
    xc.wait()


def kernel(gum, gim):
    n, d = gum.shape
    at = gum.T
    bt = gim.T
    x, aot, bot = pl.pallas_call(
        _pipe_kernel,
        in_specs=[
            pl.BlockSpec(memory_space=pl.ANY),
            pl.BlockSpec(memory_space=pl.ANY),
        ],
        out_specs=(
            pl.BlockSpec(memory_space=pl.ANY),
            pl.BlockSpec(memory_space=pl.ANY),
            pl.BlockSpec(memory_space=pl.ANY),
        ),
        out_shape=(
            jax.ShapeDtypeStruct((n,), jnp.float32),
            jax.ShapeDtypeStruct((d, n), jnp.float32),
            jax.ShapeDtypeStruct((d, n), jnp.float32),
        ),
        scratch_shapes=[
            pltpu.VMEM((_D, _N), jnp.float32),
            pltpu.VMEM((_D, _N), jnp.float32),
            pltpu.VMEM((_N,), jnp.float32),
            pltpu.SemaphoreType.DMA((_NCH, 2)),
            pltpu.SemaphoreType.DMA((_NCH, 2)),
            pltpu.SemaphoreType.DMA,
        ],
    )(at, bt)
    return (x, aot.T, bot.T)
